# baseline (speedup 1.0000x reference)
import jax
import jax.numpy as jnp
from jax import lax
from jax.experimental import pallas as pl
from jax.experimental.pallas import tpu as pltpu
from jax.experimental.pallas import tpu_sc as plsc

_B = 2097152
_NC = 2
_NS = 16
_NW = _NC * _NS
_PER_W = _B // _NW
_C = 4096
_NCHUNK = _PER_W // _C
_DG = _C // 32
_TAB = 1024 * 1024
_TAB_SLICE = _TAB // _NS

_GDN = lax.GatherDimensionNumbers(
    offset_dims=(), collapsed_slice_dims=(0,), start_index_map=(0,))


def _shuffle(v, idx):
    return lax.gather(v, idx[:, None], _GDN, slice_sizes=(1,),
                      mode=lax.GatherScatterMode.PROMISE_IN_BOUNDS)


def _sc_body(inp_hbm, table_hbm, bm_hbm, out_hbm,
             in_v0, in_v1, in_v2, idx_v0, idx_v1, idx_v2,
             gat_v0, gat_v1, gat_v2, bm_v, tab_sh,
             sem_i, sem_g, sem_o):
    wid = lax.axis_index("s") * _NC + lax.axis_index("c")
    pltpu.sync_copy(bm_hbm, bm_v)
    b0 = bm_v[pl.ds(0, 16)]
    b1 = bm_v[pl.ds(16, 16)]
    m0 = bm_v[pl.ds(32, 16)]
    m1 = bm_v[pl.ds(48, 16)]
    lane = lax.iota(jnp.int32, 16)
    pe = (2 * lane) & 15
    po = pe + 1
    lo_half = lane < 8
    w0 = wid * _PER_W
    in_b = (in_v0, in_v1, in_v2)
    idx_b = (idx_v0, idx_v1, idx_v2)
    gat_b = (gat_v0, gat_v1, gat_v2)

    def in_copy(ci):
        p = ci % 3
        s = pl.ds(2 * (w0 + ci * _C), 2 * _C)
        return pltpu.make_async_copy(inp_hbm.at[s], in_b[p], sem_i.at[p])

    def gather_copy(ci):
        p = ci % 3
        return pltpu.make_async_copy(tab_sh.at[idx_b[p]], gat_b[p],
                                     sem_g.at[p])

    def out_copy(ci):
        p = ci % 3
        return pltpu.make_async_copy(gat_b[p],
                                     out_hbm.at[pl.ds(w0 + ci * _C, _C)],
                                     sem_o.at[p])

    def compute(ci):
        in_r, idx_r = in_b[ci % 3], idx_b[ci % 3]

        def quantize(pos, vel):
            row = ((pos + b0) * m0).astype(jnp.int32)
            col = ((vel + b1) * m1).astype(jnp.int32)
            return row * 1024 + col

        def dg_body(g, c2):
            base = g * 64
            for half in range(2):
                va = in_r[pl.ds(base + half * 32, 16)]
                vb = in_r[pl.ds(base + half * 32 + 16, 16)]
                pos = jnp.where(lo_half, _shuffle(va, pe), _shuffle(vb, pe))
                vel = jnp.where(lo_half, _shuffle(va, po), _shuffle(vb, po))
                idx_r[pl.ds(g * 32 + half * 16, 16)] = quantize(pos, vel)
            return c2

        lax.fori_loop(0, _DG, dg_body, 0, unroll=4)

    sid = lax.axis_index("s")
    for t in range(_NS):
        @pl.when(sid == t)
        def _stage(t=t):
            ts = pl.ds(t * _TAB_SLICE, _TAB_SLICE)
            pltpu.sync_copy(table_hbm.at[ts], tab_sh.at[ts])
    plsc.subcore_barrier()
    for cj in range(3):
        in_copy(cj).start()
    for ci in range(_NCHUNK):
        in_copy(ci).wait()
        compute(ci)
        if ci + 3 < _NCHUNK:
            in_copy(ci + 3).start()
        if ci >= 1:
            gather_copy(ci - 1).wait()
            out_copy(ci - 1).start()
        if ci >= 2:
            out_copy(ci - 2).wait()
        gather_copy(ci).start()
    gather_copy(_NCHUNK - 1).wait()
    out_copy(_NCHUNK - 1).start()
    out_copy(_NCHUNK - 2).wait()
    out_copy(_NCHUNK - 1).wait()


def kernel(inp, data, b, m):
    inp_flat = inp.reshape(-1)
    table = data.reshape(-1)
    bm = jnp.concatenate([
        jnp.broadcast_to(b[0], (16,)),
        jnp.broadcast_to(b[1], (16,)),
        jnp.broadcast_to(m[0], (16,)),
        jnp.broadcast_to(m[1], (16,)),
    ]).astype(jnp.float32)
    mesh = plsc.VectorSubcoreMesh(core_axis_name="c", subcore_axis_name="s",
                                  num_cores=_NC)
    return pl.kernel(
        _sc_body,
        out_type=jax.ShapeDtypeStruct((_B,), jnp.float32),
        mesh=mesh,
        scratch_types=[
            pltpu.VMEM((2 * _C,), jnp.float32),
            pltpu.VMEM((2 * _C,), jnp.float32),
            pltpu.VMEM((2 * _C,), jnp.float32),
            pltpu.VMEM((_C,), jnp.int32),
            pltpu.VMEM((_C,), jnp.int32),
            pltpu.VMEM((_C,), jnp.int32),
            pltpu.VMEM((_C,), jnp.float32),
            pltpu.VMEM((_C,), jnp.float32),
            pltpu.VMEM((_C,), jnp.float32),
            pltpu.VMEM((64,), jnp.float32),
            pltpu.VMEM_SHARED((_TAB,), jnp.float32),
            pltpu.SemaphoreType.DMA((3,)),
            pltpu.SemaphoreType.DMA((3,)),
            pltpu.SemaphoreType.DMA((3,)),
        ],
    )(inp_flat, table, bm)


# speedup vs baseline: 27.5459x; 27.5459x over previous
import jax
import jax.numpy as jnp
from jax import lax
from jax.experimental import pallas as pl
from jax.experimental.pallas import tpu as pltpu
from jax.experimental.pallas import tpu_sc as plsc

_B = 2097152
_NC = 2
_NS = 16
_NW = _NC * _NS
_PER_W = _B // _NW
_C = 4096
_NCHUNK = _PER_W // _C
_G = _C // 16
_TAB = 1024 * 1024
_TAB_SLICE = _TAB // _NS


def _sc_body(pos_hbm, vel_hbm, table_hbm, bm_hbm, out_hbm,
             pos_v0, pos_v1, pos_v2, vel_v0, vel_v1, vel_v2,
             idx_v0, idx_v1, idx_v2, gat_v0, gat_v1, gat_v2,
             bm_v, tab_sh, sem_p, sem_v, sem_g, sem_o):
    wid = lax.axis_index("s") * _NC + lax.axis_index("c")
    pltpu.sync_copy(bm_hbm, bm_v)
    b0 = bm_v[pl.ds(0, 16)]
    b1 = bm_v[pl.ds(16, 16)]
    m0 = bm_v[pl.ds(32, 16)]
    m1 = bm_v[pl.ds(48, 16)]
    w0 = wid * _PER_W
    pos_b = (pos_v0, pos_v1, pos_v2)
    vel_b = (vel_v0, vel_v1, vel_v2)
    idx_b = (idx_v0, idx_v1, idx_v2)
    gat_b = (gat_v0, gat_v1, gat_v2)

    def in_copies(ci):
        p = ci % 3
        s = pl.ds(w0 + ci * _C, _C)
        cp_p = pltpu.make_async_copy(pos_hbm.at[s], pos_b[p], sem_p.at[p])
        cp_v = pltpu.make_async_copy(vel_hbm.at[s], vel_b[p], sem_v.at[p])
        return cp_p, cp_v

    def gather_copy(ci):
        p = ci % 3
        return pltpu.make_async_copy(tab_sh.at[idx_b[p]], gat_b[p],
                                     sem_g.at[p])

    def out_copy(ci):
        p = ci % 3
        return pltpu.make_async_copy(gat_b[p],
                                     out_hbm.at[pl.ds(w0 + ci * _C, _C)],
                                     sem_o.at[p])

    def compute(ci):
        p = ci % 3
        pos_r, vel_r, idx_r = pos_b[p], vel_b[p], idx_b[p]

        def group_body(g, c2):
            s = pl.ds(g * 16, 16)
            pos = pos_r[s]
            vel = vel_r[s]
            row = ((pos + b0) * m0).astype(jnp.int32)
            col = ((vel + b1) * m1).astype(jnp.int32)
            idx_r[s] = row * 1024 + col
            return c2

        lax.fori_loop(0, _G, group_body, 0, unroll=8)

    sid = lax.axis_index("s")
    for t in range(_NS):
        @pl.when(sid == t)
        def _stage(t=t):
            ts = pl.ds(t * _TAB_SLICE, _TAB_SLICE)
            pltpu.sync_copy(table_hbm.at[ts], tab_sh.at[ts])
    plsc.subcore_barrier()
    for cj in range(3):
        cp_p, cp_v = in_copies(cj)
        cp_p.start()
        cp_v.start()
    for ci in range(_NCHUNK):
        cp_p, cp_v = in_copies(ci)
        cp_p.wait()
        cp_v.wait()
        compute(ci)
        if ci + 3 < _NCHUNK:
            cp_p, cp_v = in_copies(ci + 3)
            cp_p.start()
            cp_v.start()
        if ci >= 1:
            gather_copy(ci - 1).wait()
            out_copy(ci - 1).start()
        if ci >= 2:
            out_copy(ci - 2).wait()
        gather_copy(ci).start()
    gather_copy(_NCHUNK - 1).wait()
    out_copy(_NCHUNK - 1).start()
    out_copy(_NCHUNK - 2).wait()
    out_copy(_NCHUNK - 1).wait()


def kernel(inp, data, b, m):
    pos = inp[:, 0]
    vel = inp[:, 1]
    table = data.reshape(-1)
    bm = jnp.concatenate([
        jnp.broadcast_to(b[0], (16,)),
        jnp.broadcast_to(b[1], (16,)),
        jnp.broadcast_to(m[0], (16,)),
        jnp.broadcast_to(m[1], (16,)),
    ]).astype(jnp.float32)
    mesh = plsc.VectorSubcoreMesh(core_axis_name="c", subcore_axis_name="s",
                                  num_cores=_NC)
    return pl.kernel(
        _sc_body,
        out_type=jax.ShapeDtypeStruct((_B,), jnp.float32),
        mesh=mesh,
        scratch_types=[
            pltpu.VMEM((_C,), jnp.float32),
            pltpu.VMEM((_C,), jnp.float32),
            pltpu.VMEM((_C,), jnp.float32),
            pltpu.VMEM((_C,), jnp.float32),
            pltpu.VMEM((_C,), jnp.float32),
            pltpu.VMEM((_C,), jnp.float32),
            pltpu.VMEM((_C,), jnp.int32),
            pltpu.VMEM((_C,), jnp.int32),
            pltpu.VMEM((_C,), jnp.int32),
            pltpu.VMEM((_C,), jnp.float32),
            pltpu.VMEM((_C,), jnp.float32),
            pltpu.VMEM((_C,), jnp.float32),
            pltpu.VMEM((64,), jnp.float32),
            pltpu.VMEM_SHARED((_TAB,), jnp.float32),
            pltpu.SemaphoreType.DMA((3,)),
            pltpu.SemaphoreType.DMA((3,)),
            pltpu.SemaphoreType.DMA((3,)),
            pltpu.SemaphoreType.DMA((3,)),
        ],
    )(pos, vel, table, bm)


# speedup vs baseline: 27.8740x; 1.0119x over previous
import jax
import jax.numpy as jnp
from jax import lax
from jax.experimental import pallas as pl
from jax.experimental.pallas import tpu as pltpu
from jax.experimental.pallas import tpu_sc as plsc

_B = 2097152
_NC = 2
_NS = 16
_NW = _NC * _NS
_PER_W = _B // _NW
_C = 4096
_NCHUNK = _PER_W // _C
_G = _C // 16
_TAB = 1024 * 1024
_TAB_SLICE = _TAB // _NS


def _sc_body(pos_hbm, vel_hbm, table_hbm, bm_hbm, out_hbm,
             pos_v0, pos_v1, pos_v2, vel_v0, vel_v1, vel_v2,
             idx_v0, idx_v1, idx_v2, gat_v0, gat_v1, gat_v2,
             bm_v, tab_sh, sem_p, sem_v, sem_g, sem_o):
    wid = lax.axis_index("s") * _NC + lax.axis_index("c")
    pltpu.sync_copy(bm_hbm, bm_v)
    b0 = bm_v[pl.ds(0, 16)]
    b1 = bm_v[pl.ds(16, 16)]
    m0 = bm_v[pl.ds(32, 16)]
    m1 = bm_v[pl.ds(48, 16)]
    w0 = wid * _PER_W
    pos_b = (pos_v0, pos_v1, pos_v2)
    vel_b = (vel_v0, vel_v1, vel_v2)
    idx_b = (idx_v0, idx_v1, idx_v2)
    gat_b = (gat_v0, gat_v1, gat_v2)

    def in_copies(ci):
        p = ci % 3
        s = pl.ds(w0 + ci * _C, _C)
        cp_p = pltpu.make_async_copy(pos_hbm.at[s], pos_b[p], sem_p.at[p])
        cp_v = pltpu.make_async_copy(vel_hbm.at[s], vel_b[p], sem_v.at[p])
        return cp_p, cp_v

    def gather_copy(ci):
        p = ci % 3
        return pltpu.make_async_copy(tab_sh.at[idx_b[p]], gat_b[p],
                                     sem_g.at[p])

    def out_copy(ci):
        p = ci % 3
        return pltpu.make_async_copy(gat_b[p],
                                     out_hbm.at[pl.ds(w0 + ci * _C, _C)],
                                     sem_o.at[p])

    def compute(ci):
        p = ci % 3
        pos_r, vel_r, idx_r = pos_b[p], vel_b[p], idx_b[p]

        def group_body(g, c2):
            s = pl.ds(g * 16, 16)
            pos = pos_r[s]
            vel = vel_r[s]
            row = ((pos + b0) * m0).astype(jnp.int32)
            col = ((vel + b1) * m1).astype(jnp.int32)
            idx_r[s] = row * 1024 + col
            return c2

        lax.fori_loop(0, _G, group_body, 0, unroll=8)

    for cj in range(3):
        cp_p, cp_v = in_copies(cj)
        cp_p.start()
        cp_v.start()
    sid = lax.axis_index("s")
    for t in range(_NS):
        @pl.when(sid == t)
        def _stage(t=t):
            ts = pl.ds(t * _TAB_SLICE, _TAB_SLICE)
            pltpu.sync_copy(table_hbm.at[ts], tab_sh.at[ts])
    plsc.subcore_barrier()
    for ci in range(_NCHUNK):
        cp_p, cp_v = in_copies(ci)
        cp_p.wait()
        cp_v.wait()
        compute(ci)
        if ci + 3 < _NCHUNK:
            cp_p, cp_v = in_copies(ci + 3)
            cp_p.start()
            cp_v.start()
        if ci >= 1:
            gather_copy(ci - 1).wait()
            out_copy(ci - 1).start()
        if ci >= 2:
            out_copy(ci - 2).wait()
        gather_copy(ci).start()
    gather_copy(_NCHUNK - 1).wait()
    out_copy(_NCHUNK - 1).start()
    out_copy(_NCHUNK - 2).wait()
    out_copy(_NCHUNK - 1).wait()


def kernel(inp, data, b, m):
    pos = inp[:, 0]
    vel = inp[:, 1]
    table = data.reshape(-1)
    bm = jnp.concatenate([
        jnp.broadcast_to(b[0], (16,)),
        jnp.broadcast_to(b[1], (16,)),
        jnp.broadcast_to(m[0], (16,)),
        jnp.broadcast_to(m[1], (16,)),
    ]).astype(jnp.float32)
    mesh = plsc.VectorSubcoreMesh(core_axis_name="c", subcore_axis_name="s",
                                  num_cores=_NC)
    return pl.kernel(
        _sc_body,
        out_type=jax.ShapeDtypeStruct((_B,), jnp.float32),
        mesh=mesh,
        scratch_types=[
            pltpu.VMEM((_C,), jnp.float32),
            pltpu.VMEM((_C,), jnp.float32),
            pltpu.VMEM((_C,), jnp.float32),
            pltpu.VMEM((_C,), jnp.float32),
            pltpu.VMEM((_C,), jnp.float32),
            pltpu.VMEM((_C,), jnp.float32),
            pltpu.VMEM((_C,), jnp.int32),
            pltpu.VMEM((_C,), jnp.int32),
            pltpu.VMEM((_C,), jnp.int32),
            pltpu.VMEM((_C,), jnp.float32),
            pltpu.VMEM((_C,), jnp.float32),
            pltpu.VMEM((_C,), jnp.float32),
            pltpu.VMEM((64,), jnp.float32),
            pltpu.VMEM_SHARED((_TAB,), jnp.float32),
            pltpu.SemaphoreType.DMA((3,)),
            pltpu.SemaphoreType.DMA((3,)),
            pltpu.SemaphoreType.DMA((3,)),
            pltpu.SemaphoreType.DMA((3,)),
        ],
    )(pos, vel, table, bm)


# speedup vs baseline: 35.0489x; 1.2574x over previous
import jax
import jax.numpy as jnp
from jax import lax
from jax.experimental import pallas as pl
from jax.experimental.pallas import tpu as pltpu
from jax.experimental.pallas import tpu_sc as plsc

_B = 2097152
_NC = 2
_NS = 16
_NW = _NC * _NS
_PER_W = _B // _NW
_C = 4096
_NCHUNK = _PER_W // _C
_G = _C // 16
_TAB = 1024 * 1024
_TAB_SLICE = _TAB // _NS


def _sc_body(pv_hbm, table_hbm, bm_hbm, out_hbm,
             pos_v0, pos_v1, pos_v2, vel_v0, vel_v1, vel_v2,
             idx_v0, idx_v1, idx_v2, gat_v0, gat_v1, gat_v2,
             bm_v, tab_sh, sem_p, sem_v, sem_g, sem_o):
    wid = lax.axis_index("s") * _NC + lax.axis_index("c")
    pltpu.sync_copy(bm_hbm, bm_v)
    b0 = bm_v[pl.ds(0, 16)]
    b1 = bm_v[pl.ds(16, 16)]
    m0 = bm_v[pl.ds(32, 16)]
    m1 = bm_v[pl.ds(48, 16)]
    w0 = wid * _PER_W
    pos_b = (pos_v0, pos_v1, pos_v2)
    vel_b = (vel_v0, vel_v1, vel_v2)
    idx_b = (idx_v0, idx_v1, idx_v2)
    gat_b = (gat_v0, gat_v1, gat_v2)

    def in_copies(ci):
        p = ci % 3
        s = pl.ds(w0 + ci * _C, _C)
        cp_p = pltpu.make_async_copy(pv_hbm.at[0, s], pos_b[p], sem_p.at[p])
        cp_v = pltpu.make_async_copy(pv_hbm.at[1, s], vel_b[p], sem_v.at[p])
        return cp_p, cp_v

    def gather_copy(ci):
        p = ci % 3
        return pltpu.make_async_copy(tab_sh.at[idx_b[p]], gat_b[p],
                                     sem_g.at[p])

    def out_copy(ci):
        p = ci % 3
        return pltpu.make_async_copy(gat_b[p],
                                     out_hbm.at[pl.ds(w0 + ci * _C, _C)],
                                     sem_o.at[p])

    def compute(ci):
        p = ci % 3
        pos_r, vel_r, idx_r = pos_b[p], vel_b[p], idx_b[p]

        def group_body(g, c2):
            s = pl.ds(g * 16, 16)
            pos = pos_r[s]
            vel = vel_r[s]
            row = ((pos + b0) * m0).astype(jnp.int32)
            col = ((vel + b1) * m1).astype(jnp.int32)
            idx_r[s] = row * 1024 + col
            return c2

        lax.fori_loop(0, _G, group_body, 0, unroll=8)

    for cj in range(3):
        cp_p, cp_v = in_copies(cj)
        cp_p.start()
        cp_v.start()
    sid = lax.axis_index("s")
    for t in range(_NS):
        @pl.when(sid == t)
        def _stage(t=t):
            ts = pl.ds(t * _TAB_SLICE, _TAB_SLICE)
            pltpu.sync_copy(table_hbm.at[ts], tab_sh.at[ts])
    plsc.subcore_barrier()
    for ci in range(_NCHUNK):
        cp_p, cp_v = in_copies(ci)
        cp_p.wait()
        cp_v.wait()
        compute(ci)
        if ci + 3 < _NCHUNK:
            cp_p, cp_v = in_copies(ci + 3)
            cp_p.start()
            cp_v.start()
        if ci >= 1:
            gather_copy(ci - 1).wait()
            out_copy(ci - 1).start()
        if ci >= 2:
            out_copy(ci - 2).wait()
        gather_copy(ci).start()
    gather_copy(_NCHUNK - 1).wait()
    out_copy(_NCHUNK - 1).start()
    out_copy(_NCHUNK - 2).wait()
    out_copy(_NCHUNK - 1).wait()


def kernel(inp, data, b, m):
    pv = inp.T
    table = data.reshape(-1)
    bm = jnp.concatenate([
        jnp.broadcast_to(b[0], (16,)),
        jnp.broadcast_to(b[1], (16,)),
        jnp.broadcast_to(m[0], (16,)),
        jnp.broadcast_to(m[1], (16,)),
    ]).astype(jnp.float32)
    mesh = plsc.VectorSubcoreMesh(core_axis_name="c", subcore_axis_name="s",
                                  num_cores=_NC)
    return pl.kernel(
        _sc_body,
        out_type=jax.ShapeDtypeStruct((_B,), jnp.float32),
        mesh=mesh,
        scratch_types=[
            pltpu.VMEM((_C,), jnp.float32),
            pltpu.VMEM((_C,), jnp.float32),
            pltpu.VMEM((_C,), jnp.float32),
            pltpu.VMEM((_C,), jnp.float32),
            pltpu.VMEM((_C,), jnp.float32),
            pltpu.VMEM((_C,), jnp.float32),
            pltpu.VMEM((_C,), jnp.int32),
            pltpu.VMEM((_C,), jnp.int32),
            pltpu.VMEM((_C,), jnp.int32),
            pltpu.VMEM((_C,), jnp.float32),
            pltpu.VMEM((_C,), jnp.float32),
            pltpu.VMEM((_C,), jnp.float32),
            pltpu.VMEM((64,), jnp.float32),
            pltpu.VMEM_SHARED((_TAB,), jnp.float32),
            pltpu.SemaphoreType.DMA((3,)),
            pltpu.SemaphoreType.DMA((3,)),
            pltpu.SemaphoreType.DMA((3,)),
            pltpu.SemaphoreType.DMA((3,)),
        ],
    )(pv, table, bm)
